# Initial kernel scaffold; baseline (speedup 1.0000x reference)
import jax
import jax.numpy as jnp
from jax.experimental import pallas as pl


def kernel(Z, emb_table):
    raise NotImplementedError("write your pallas kernel here")


# speedup vs baseline: 1.2890x; 1.2890x over previous
import functools

import jax
import jax.numpy as jnp
from jax import lax
from jax.experimental import pallas as pl
from jax.experimental.pallas import tpu as pltpu
from jax.experimental.pallas import tpu_sc as plsc

TOTAL_DIM = 366
TAB_DIM = 128
N_INV = 87

NC = 2
NS = 16
NW = NC * NS

CHUNK = 128


def _body(table_hbm, idx_hbm, zeros_hbm, out_hbm, idx_v, block_v, sem):
    wid = lax.axis_index("s") * NC + lax.axis_index("c")
    n = out_hbm.shape[0]
    rows_per_w = n // NW
    steps = rows_per_w // CHUNK
    base = wid * rows_per_w

    pltpu.sync_copy(zeros_hbm, block_v)

    def step(j, carry):
        b = base + j * CHUNK
        pltpu.sync_copy(idx_hbm.at[pl.ds(b, CHUNK)], idx_v)
        pltpu.async_copy(
            table_hbm.at[idx_v], block_v.at[:, pl.ds(0, TAB_DIM)], sem
        ).wait()
        pltpu.sync_copy(block_v, out_hbm.at[pl.ds(b, CHUNK)])
        return carry

    lax.fori_loop(0, steps, step, 0)


@jax.jit
def kernel(Z, emb_table):
    n = Z.shape[0]
    padded = jnp.zeros((N_INV, TAB_DIM), jnp.float32).at[:, :N_INV].set(emb_table)
    idx = Z.reshape(n)
    zeros_blk = jnp.zeros((CHUNK, TOTAL_DIM), jnp.float32)

    mesh = plsc.VectorSubcoreMesh(core_axis_name="c", subcore_axis_name="s")
    run = pl.kernel(
        _body,
        out_type=jax.ShapeDtypeStruct((n, TOTAL_DIM), jnp.float32),
        mesh=mesh,
        scratch_types=[
            pltpu.VMEM((CHUNK,), jnp.int32),
            pltpu.VMEM((CHUNK, TOTAL_DIM), jnp.float32),
            pltpu.SemaphoreType.DMA,
        ],
    )
    return run(padded, idx, zeros_blk)
